# Initial kernel scaffold; baseline (speedup 1.0000x reference)
import jax
import jax.numpy as jnp
from jax.experimental import pallas as pl


def kernel(x, edge_index, batch, edge_attr, neg_edge_index, params):
    raise NotImplementedError("write your pallas kernel here")


# speedup vs baseline: 13.3075x; 13.3075x over previous
import jax
import jax.numpy as jnp
from jax import lax
from jax.experimental import pallas as pl
from jax.experimental.pallas import tpu as pltpu
from jax.experimental.pallas import tpu_sc as plsc

N = 50000
B = 50
NPG = 1000
E = 800000
H = 64
EMD = 200
IN_CH = 9
OUT = 10
EPS = 1e-15
HALF = N // 2
CH = 128

ENC_CHUNKS = 13
ENC_PT = ENC_CHUNKS * CH
NP_ENC = 32 * ENC_PT

SP_PT = 1600
SP_ROWS = 16 * SP_PT
EP = E + 2 * CH

LG_GRP = 16
LG_GROUPS = 25
LG_PT = LG_GROUPS * LG_GRP * CH
NPL = 32 * LG_PT

BLKL = 16000
NBL = 2 * E // BLKL
BLKD = 2000
NBD = N // BLKD

_mesh = plsc.VectorSubcoreMesh(core_axis_name="c", subcore_axis_name="s")
_sc_params = pltpu.CompilerParams(use_tc_tiling_on_sc=False,
                                  needs_layout_passes=False)


def _enc_body(xiT, emb, out, ibuf, gbuf, hbuf, isem, gsem):
    c = lax.axis_index("c")
    sid = lax.axis_index("s")
    wid = sid * 2 + c
    for ch in range(ENC_CHUNKS):
        nb = wid * ENC_PT + ch * CH
        dsc = [pltpu.async_copy(xiT.at[i, pl.ds(nb, CH)], ibuf.at[i], isem)
               for i in range(IN_CH)]
        for d in dsc:
            d.wait()
        gds = [pltpu.async_copy(emb.at[ibuf.at[i]], gbuf.at[i], gsem)
               for i in range(IN_CH)]
        for d in gds:
            d.wait()

        def body(e4, _):
            for u in range(4):
                e = e4 * 4 + u
                for jv in range(4):
                    sl = pl.ds(jv * 16, 16)
                    acc = gbuf[0, e, sl]
                    for i in range(1, IN_CH):
                        acc = acc + gbuf[i, e, sl]
                    hbuf[e, sl] = acc
            return 0

        lax.fori_loop(0, CH // 4, body, 0)
        pltpu.sync_copy(hbuf, out.at[pl.ds(nb, CH)])


_enc = pl.kernel(
    _enc_body,
    out_type=jax.ShapeDtypeStruct((NP_ENC, H), jnp.float32),
    mesh=_mesh,
    compiler_params=_sc_params,
    scratch_types=[
        pltpu.VMEM((IN_CH, CH), jnp.int32),
        pltpu.VMEM((IN_CH, CH, H), jnp.float32),
        pltpu.VMEM((CH, H), jnp.float32),
        pltpu.SemaphoreType.DMA,
        pltpu.SemaphoreType.DMA,
    ],
)


def _mp_body(T, gidxh, didxh, espl, zer, out, mbuf, gidx, didx, esv, aggsh,
             i0, i1, i2, i3, i4, i5, g0, g1, g2, s0, s1, s2):
    isems = [i0, i1, i2, i3, i4, i5]
    gsems = [g0, g1, g2]
    ssems = [s0, s1, s2]
    c = lax.axis_index("c")
    sid = lax.axis_index("s")

    pltpu.sync_copy(zer, aggsh.at[pl.ds(sid * SP_PT, SP_PT)])
    pltpu.sync_copy(espl, esv)
    lanes = lax.iota(jnp.int32, 16)
    esplit_f = jnp.sum(jnp.where(lanes == 0, esv[...], jnp.float32(0.0)))
    esplit = esplit_f.astype(jnp.int32)
    lo = jnp.where(c == 0, 0, esplit)
    hi = jnp.where(c == 0, esplit, E)
    per = (hi - lo + 15) // 16
    start = lo + sid * per
    end = jnp.minimum(start + per, hi)
    start8 = (start // 8) * 8
    nch = jnp.maximum((end - start8 + CH - 1) // CH, 0)
    ngrp = (nch + 7) // 6

    plsc.subcore_barrier()

    def fetch_idx(k, b6):
        base = start8 + k * CH
        pltpu.async_copy(gidxh.at[pl.ds(base, CH)], gidx.at[b6], isems[b6])
        pltpu.async_copy(didxh.at[pl.ds(base, CH)], didx.at[b6], isems[b6])

    def wait_idx(b6):
        pltpu.make_async_copy(gidxh.at[pl.ds(0, CH)], gidx.at[b6],
                              isems[b6]).wait()
        pltpu.make_async_copy(didxh.at[pl.ds(0, CH)], didx.at[b6],
                              isems[b6]).wait()

    def issue_gather(b6, b3):
        pltpu.async_copy(T.at[gidx.at[b6]], mbuf.at[b3], gsems[b3])

    def wait_gather(b6, b3):
        pltpu.make_async_copy(T.at[gidx.at[b6]], mbuf.at[b3],
                              gsems[b3]).wait()

    def issue_scatter(b6, b3):
        pltpu.async_copy(mbuf.at[b3], aggsh.at[didx.at[b6]], ssems[b3],
                         add=True)

    def wait_scatter(b6, b3):
        pltpu.make_async_copy(mbuf.at[b3], aggsh.at[didx.at[b6]],
                              ssems[b3]).wait()

    def relu_chunk(k, b3):
        base = start8 + k * CH

        def body(e4, _):
            for u in range(4):
                e = e4 * 4 + u
                eg = base + e
                vf = jnp.where((eg >= start) & (eg < end),
                               jnp.float32(1.0), jnp.float32(0.0))
                for jv in range(4):
                    sl = pl.ds(jv * 16, 16)
                    mbuf[b3, e, sl] = jnp.maximum(mbuf[b3, e, sl], 0.0) * vf
            return 0

        lax.fori_loop(0, CH // 4, body, 0)

    @pl.when(nch > 0)
    def _():
        fetch_idx(0, 0)

    @pl.when(nch > 1)
    def _():
        fetch_idx(1, 1)

    def grp_body(grp, _):
        for b6 in range(6):
            k = grp * 6 + b6
            b3 = b6 % 3
            pb6 = (b6 + 5) % 6
            pb3 = (b3 + 2) % 3

            @pl.when(k < nch)
            def _():
                @pl.when(k >= 3)
                def _():
                    wait_scatter((b6 + 3) % 6, b3)

                wait_idx(b6)
                issue_gather(b6, b3)

            @pl.when((k >= 1) & (k <= nch))
            def _():
                wait_gather(pb6, pb3)
                relu_chunk(k - 1, pb3)
                issue_scatter(pb6, pb3)

            @pl.when(k + 2 < nch)
            def _():
                fetch_idx(k + 2, (b6 + 2) % 6)
        return 0

    lax.fori_loop(0, ngrp, grp_body, 0)

    for b in range(3):
        @pl.when(nch > b)
        def _():
            wait_scatter(b, b)

    plsc.subcore_barrier()
    pltpu.sync_copy(aggsh.at[pl.ds(sid * SP_PT, SP_PT)],
                    out.at[c, pl.ds(sid * SP_PT, SP_PT)])


_mp = pl.kernel(
    _mp_body,
    out_type=jax.ShapeDtypeStruct((2, SP_ROWS, H), jnp.float32),
    mesh=_mesh,
    compiler_params=_sc_params,
    scratch_types=[
        pltpu.VMEM((3, CH, H), jnp.float32),
        pltpu.VMEM((6, CH), jnp.int32),
        pltpu.VMEM((6, CH), jnp.int32),
        pltpu.VMEM((16,), jnp.float32),
        pltpu.VMEM_SHARED((SP_ROWS, H), jnp.float32),
    ] + [pltpu.SemaphoreType.DMA] * 12,
)


def _lg_body(Df, idxh, out, ibuf, dbuf, iA, iB, gsem, dA, dB):
    c = lax.axis_index("c")
    sid = lax.axis_index("s")
    wid = sid * 2 + c
    rbase = wid * (LG_PT // CH)
    isems = [iA, iB]
    dsems = [dA, dB]

    pend_idx = [None, None]
    pend_drain = [None, None]
    pend_idx[0] = pltpu.async_copy(idxh.at[pl.ds(rbase, LG_GRP)],
                                   ibuf.at[0], isems[0])
    pend_idx[1] = pltpu.async_copy(idxh.at[pl.ds(rbase + LG_GRP, LG_GRP)],
                                   ibuf.at[1], isems[1])
    for grp in range(LG_GROUPS):
        b = grp % 2
        rb = rbase + grp * LG_GRP
        pend_idx[b].wait()
        if pend_drain[b] is not None:
            pend_drain[b].wait()
        gds = [pltpu.async_copy(Df.at[ibuf.at[b, j]], dbuf.at[b, j], gsem)
               for j in range(LG_GRP)]
        for gd in gds:
            gd.wait()
        if grp + 2 < LG_GROUPS:
            pend_idx[b] = pltpu.async_copy(
                idxh.at[pl.ds(rbase + (grp + 2) * LG_GRP, LG_GRP)],
                ibuf.at[b], isems[b])
        pend_drain[b] = pltpu.async_copy(dbuf.at[b],
                                         out.at[pl.ds(rb, LG_GRP)], dsems[b])
    for b in range(2):
        if pend_drain[b] is not None:
            pend_drain[b].wait()


_lg = pl.kernel(
    _lg_body,
    out_type=jax.ShapeDtypeStruct((NPL // CH, CH), jnp.float32),
    mesh=_mesh,
    compiler_params=_sc_params,
    scratch_types=[
        pltpu.VMEM((2, LG_GRP, CH), jnp.int32),
        pltpu.VMEM((2, LG_GRP, CH), jnp.float32),
    ] + [pltpu.SemaphoreType.DMA] * 5,
)


def _buildT_body(hr, e1r, dvr, Tout):
    hv = hr[...]
    for k in range(4):
        Tout[k] = hv + e1r[...] + k * dvr[...]


def _buildT(h, e1r, dvr):
    return pl.pallas_call(
        _buildT_body,
        grid=(NBD,),
        in_specs=[
            pl.BlockSpec((BLKD, H), lambda j: (j, 0)),
            pl.BlockSpec((1, H), lambda j: (0, 0)),
            pl.BlockSpec((1, H), lambda j: (0, 0)),
        ],
        out_specs=pl.BlockSpec((4, BLKD, H), lambda j: (0, j, 0)),
        out_shape=jax.ShapeDtypeStruct((4, N, H), jnp.float32),
    )(h, e1r, dvr)


def _dense_call(emit_T, relu_out):
    def body(*args):
        if emit_T:
            hc, agg, W1, b1, W2, b2, epsr, e1r, dvr, hout, Tout = args
        else:
            hc, agg, W1, b1, W2, b2, epsr, hout = args
        z = (1.0 + epsr[0, 0]) * hc[...] + agg[...]
        z = jnp.maximum(
            jnp.dot(z, W1[...], preferred_element_type=jnp.float32) + b1[...],
            0.0)
        z = jnp.dot(z, W2[...], preferred_element_type=jnp.float32) + b2[...]
        if relu_out:
            z = jnp.maximum(z, 0.0)
        hout[...] = z
        if emit_T:
            for k in range(4):
                Tout[k] = z + e1r[...] + k * dvr[...]

    blk = pl.BlockSpec((BLKD, H), lambda j: (j, 0))
    wspec = pl.BlockSpec((H, H), lambda j: (0, 0))
    bspec = pl.BlockSpec((1, H), lambda j: (0, 0))
    escp = pl.BlockSpec((1, 1), lambda j: (0, 0))
    in_specs = [blk, blk, wspec, bspec, wspec, bspec, escp]
    out_specs = blk
    out_shape = jax.ShapeDtypeStruct((N, H), jnp.float32)
    if emit_T:
        in_specs = in_specs + [bspec, bspec]
        out_specs = (blk, pl.BlockSpec((4, BLKD, H), lambda j: (0, j, 0)))
        out_shape = (out_shape, jax.ShapeDtypeStruct((4, N, H), jnp.float32))
    return pl.pallas_call(
        body, grid=(NBD,), in_specs=in_specs, out_specs=out_specs,
        out_shape=out_shape)


_dense_T = _dense_call(True, True)
_dense_last = _dense_call(False, False)


def _ddot_body(xl, xg, out):
    Z = xl[...] + xg[...]
    dd = lax.dot_general(Z, Z, (((1,), (1,)), ((), ())),
                         preferred_element_type=jnp.float32)
    out[...] = dd.reshape(1, NPG, NPG)


def _ddot(xl, xg):
    return pl.pallas_call(
        _ddot_body,
        grid=(B,),
        in_specs=[
            pl.BlockSpec((NPG, H), lambda g: (g, 0)),
            pl.BlockSpec((NPG, H), lambda g: (g, 0)),
        ],
        out_specs=pl.BlockSpec((1, NPG, NPG), lambda g: (g, 0, 0)),
        out_shape=jax.ShapeDtypeStruct((B, NPG, NPG), jnp.float32),
    )(xl, xg)


def _loss_body(d, g, acc):
    pid = pl.program_id(0)

    @pl.when(pid == 0)
    def _():
        acc[...] = jnp.zeros((8, 64), jnp.float32)

    dv = d[0, 0, :]
    sg = 1.0 / (1.0 + jnp.exp(-dv))
    tpos = -jnp.log(sg + EPS)
    tneg = -jnp.log(1.0 - sg + EPS)
    posf = jnp.where(pid < NBL // 2, jnp.float32(1.0), jnp.float32(0.0))
    t = posf * tpos + (1.0 - posf) * tneg
    oh = (g[0, 0, :][:, None] == lax.broadcasted_iota(jnp.int32, (BLKL, 64), 1)
          ).astype(jnp.float32)
    rows = lax.broadcasted_iota(jnp.int32, (8, BLKL), 0)
    tb = t[None, :]
    M = jnp.where(rows == 0, tb * posf,
                  jnp.where(rows == 1, posf,
                            jnp.where(rows == 2, tb * (1.0 - posf), 0.0)))
    acc[...] += jnp.dot(M, oh, preferred_element_type=jnp.float32)


def _loss(dmat, gmat):
    return pl.pallas_call(
        _loss_body,
        grid=(NBL,),
        in_specs=[
            pl.BlockSpec((1, 1, BLKL), lambda j: (j, 0, 0)),
            pl.BlockSpec((1, 1, BLKL), lambda j: (j, 0, 0)),
        ],
        out_specs=pl.BlockSpec((8, 64), lambda j: (0, 0)),
        out_shape=jax.ShapeDtypeStruct((8, 64), jnp.float32),
    )(dmat, gmat)


def _fin_body(xg, accr, W1r, b1r, gr, br, Wc, bc, louts, recout):
    pooled = jnp.sum(xg[...].reshape(B, NPG, H), axis=1)
    o = jnp.dot(pooled, W1r[...], preferred_element_type=jnp.float32) + b1r[...]
    mu = jnp.mean(o, axis=0, keepdims=True)
    var = jnp.mean((o - mu) ** 2, axis=0, keepdims=True)
    o = gr[...] * (o - mu) / jnp.sqrt(var + 1e-5) + br[...]
    o = jnp.maximum(o, 0.0)
    louts[...] = jnp.dot(o, Wc[...], preferred_element_type=jnp.float32) + bc[...]
    a = accr[...]
    lanes = lax.broadcasted_iota(jnp.int32, (1, 64), 1)
    lossv = jnp.where(lanes < B, (a[0:1, :] + a[2:3, :]) / a[1:2, :], 0.0)
    recout[...] = jnp.sum(lossv, keepdims=True) / B


def _final(xg, acc, W1r, b1r, gr, br, Wc, bc):
    return pl.pallas_call(
        _fin_body,
        out_shape=(jax.ShapeDtypeStruct((B, 128), jnp.float32),
                   jax.ShapeDtypeStruct((1, 1), jnp.float32)),
    )(xg, acc, W1r, b1r, gr, br, Wc, bc)


def kernel(x, edge_index, batch, edge_attr, neg_edge_index, params):
    p = params
    src = edge_index[0].astype(jnp.int32)
    dst = edge_index[1].astype(jnp.int32)
    s = jnp.sum(edge_attr, axis=1).astype(jnp.int32)
    g_e = dst // NPG
    E1 = p["edge_emb"][1]
    Dv = (p["edge_emb"][2] - E1) / 3.0
    e1r = E1.reshape(1, H)
    dvr = Dv.reshape(1, H)

    xi = x.astype(jnp.int32) + jnp.arange(IN_CH, dtype=jnp.int32)[None, :] * EMD
    xiT = jnp.pad(xi.T, ((0, 0), (0, NP_ENC - N)))
    emb = p["atom_emb"].reshape(IN_CH * EMD, H)
    h = _enc(xiT, emb)[:N]

    gidx = jnp.pad(s * N + src, (0, EP - E))
    didx = jnp.pad(dst % HALF, (0, EP - E))
    esplit = jnp.searchsorted(g_e, 25).astype(jnp.float32)
    esv = jnp.zeros((16,), jnp.float32).at[0].set(esplit)
    zer = jnp.zeros((SP_PT, H), jnp.float32)

    def mp(Tfull):
        o = _mp(Tfull.reshape(4 * N, H), gidx, didx, esv, zer)
        return jnp.concatenate([o[0, :HALF], o[1, :HALF]], axis=0)

    T1 = _buildT(h, e1r, dvr)

    def run_net(plist):
        p0, p1 = plist
        agg0 = mp(T1)
        hc1, T2 = _dense_T(h, agg0, p0["W1"], p0["b1"].reshape(1, H),
                           p0["W2"], p0["b2"].reshape(1, H),
                           p0["eps"].reshape(1, 1), e1r, dvr)
        agg1 = mp(T2)
        return _dense_last(hc1, agg1, p1["W1"], p1["b1"].reshape(1, H),
                           p1["W2"], p1["b2"].reshape(1, H),
                           p1["eps"].reshape(1, 1))

    x_local = run_net(p["local"])
    x_global = run_net(p["glob"])

    Df = _ddot(x_local, x_global).reshape(B * NPG * NPG)
    pos_idx = g_e * (NPG * NPG) + (src % NPG) * NPG + (dst % NPG)
    ns = neg_edge_index[0].astype(jnp.int32)
    nd = neg_edge_index[1].astype(jnp.int32)
    neg_idx = (ns // NPG) * (NPG * NPG) + (ns % NPG) * NPG + (nd % NPG)
    idx_all = jnp.pad(jnp.concatenate([pos_idx, neg_idx]),
                      (0, NPL - 2 * E)).reshape(NPL // CH, CH)
    d_all = _lg(Df, idx_all).reshape(NPL)
    dmat = d_all[:2 * E].reshape(NBL, 1, BLKL)
    gmat = jnp.concatenate([g_e, g_e]).reshape(NBL, 1, BLKL)
    acc = _loss(dmat, gmat)

    logits_pad, rec = _final(
        x_global, acc, p["W_lin1"], p["b_lin1"].reshape(1, H),
        p["bn_gamma"].reshape(1, H), p["bn_beta"].reshape(1, H),
        jnp.pad(p["W_clf"], ((0, 0), (0, 128 - OUT))),
        jnp.pad(p["b_clf"], (0, 128 - OUT)).reshape(1, 128))
    return (logits_pad[:, :OUT], x_local, x_global, rec.reshape(()))


# speedup vs baseline: 13.6277x; 1.0241x over previous
import jax
import jax.numpy as jnp
from jax import lax
from jax.experimental import pallas as pl
from jax.experimental.pallas import tpu as pltpu
from jax.experimental.pallas import tpu_sc as plsc

N = 50000
B = 50
NPG = 1000
E = 800000
H = 64
EMD = 200
IN_CH = 9
OUT = 10
EPS = 1e-15
HALF = N // 2
CH = 128

ENC_CHUNKS = 13
ENC_PT = ENC_CHUNKS * CH
NP_ENC = 32 * ENC_PT

SP_PT = 1600
SP_ROWS = 16 * SP_PT
EP = E + 2 * CH

LG_GRP = 16
LG_GROUPS = 25
LG_PT = LG_GROUPS * LG_GRP * CH
NPL = 32 * LG_PT

BLKL = 16000
NBL = 2 * E // BLKL
BLKD = 1000
NBD = N // BLKD

_mesh = plsc.VectorSubcoreMesh(core_axis_name="c", subcore_axis_name="s")
_sc_params = pltpu.CompilerParams(use_tc_tiling_on_sc=False,
                                  needs_layout_passes=False)


ECH = 64
ENC_NCH = ENC_PT // ECH


def _enc_body(xiT, emb, out, ibuf, gbuf, hbuf, i0, i1, g0, g1, d0, d1):
    c = lax.axis_index("c")
    sid = lax.axis_index("s")
    wid = sid * 2 + c
    isems = [i0, i1]
    gsems = [g0, g1]
    dsems = [d0, d1]

    def issue_idx(k, b):
        nb = wid * ENC_PT + k * ECH
        return pltpu.async_copy(xiT.at[:, pl.ds(nb, ECH)], ibuf.at[b],
                                isems[b])

    def issue_gathers(b):
        return [pltpu.async_copy(emb.at[ibuf.at[b, i]], gbuf.at[b, i],
                                 gsems[b]) for i in range(IN_CH)]

    idxd = [issue_idx(0, 0), issue_idx(1, 1)]
    gatd = [None, None]
    hd = [None, None]
    idxd[0].wait()
    gatd[0] = issue_gathers(0)
    for k in range(ENC_NCH):
        b = k % 2
        nb = wid * ENC_PT + k * ECH
        if k + 1 < ENC_NCH:
            idxd[(k + 1) % 2].wait()
            gatd[(k + 1) % 2] = issue_gathers((k + 1) % 2)
        for d in gatd[b]:
            d.wait()
        if k + 2 < ENC_NCH:
            idxd[b] = issue_idx(k + 2, b)
        if hd[b] is not None:
            hd[b].wait()

        def body(e2, _):
            for u in range(2):
                e = e2 * 2 + u
                for jv in range(4):
                    sl = pl.ds(jv * 16, 16)
                    acc = gbuf[b, 0, e, sl]
                    for i in range(1, IN_CH):
                        acc = acc + gbuf[b, i, e, sl]
                    hbuf[b, e, sl] = acc
            return 0

        lax.fori_loop(0, ECH // 2, body, 0)
        hd[b] = pltpu.async_copy(hbuf.at[b], out.at[pl.ds(nb, ECH)], dsems[b])
    for b in range(2):
        if hd[b] is not None:
            hd[b].wait()


_enc = pl.kernel(
    _enc_body,
    out_type=jax.ShapeDtypeStruct((NP_ENC, H), jnp.float32),
    mesh=_mesh,
    compiler_params=_sc_params,
    name="sc_enc",
    scratch_types=[
        pltpu.VMEM((2, IN_CH, ECH), jnp.int32),
        pltpu.VMEM((2, IN_CH, ECH, H), jnp.float32),
        pltpu.VMEM((2, ECH, H), jnp.float32),
    ] + [pltpu.SemaphoreType.DMA] * 6,
)


def _mp_body(T, gidxh, didxh, espl, zer, out, mbuf, gidx, didx, esv, aggsh,
             i0, i1, i2, i3, i4, i5, g0, g1, g2, s0, s1, s2):
    isems = [i0, i1, i2, i3, i4, i5]
    gsems = [g0, g1, g2]
    ssems = [s0, s1, s2]
    c = lax.axis_index("c")
    sid = lax.axis_index("s")

    pltpu.sync_copy(zer, aggsh.at[pl.ds(sid * SP_PT, SP_PT)])
    pltpu.sync_copy(espl, esv)
    lanes = lax.iota(jnp.int32, 16)
    esplit_f = jnp.sum(jnp.where(lanes == 0, esv[...], jnp.float32(0.0)))
    esplit = esplit_f.astype(jnp.int32)
    lo = jnp.where(c == 0, 0, esplit)
    hi = jnp.where(c == 0, esplit, E)
    per = (hi - lo + 15) // 16
    start = lo + sid * per
    end = jnp.minimum(start + per, hi)
    start8 = (start // 8) * 8
    nch = jnp.maximum((end - start8 + CH - 1) // CH, 0)
    ngrp = (nch + 7) // 6

    plsc.subcore_barrier()

    def fetch_idx(k, b6):
        base = start8 + k * CH
        pltpu.async_copy(gidxh.at[pl.ds(base, CH)], gidx.at[b6], isems[b6])
        pltpu.async_copy(didxh.at[pl.ds(base, CH)], didx.at[b6], isems[b6])

    def wait_idx(b6):
        pltpu.make_async_copy(gidxh.at[pl.ds(0, CH)], gidx.at[b6],
                              isems[b6]).wait()
        pltpu.make_async_copy(didxh.at[pl.ds(0, CH)], didx.at[b6],
                              isems[b6]).wait()

    def issue_gather(b6, b3):
        pltpu.async_copy(T.at[gidx.at[b6]], mbuf.at[b3], gsems[b3])

    def wait_gather(b6, b3):
        pltpu.make_async_copy(T.at[gidx.at[b6]], mbuf.at[b3],
                              gsems[b3]).wait()

    def issue_scatter(b6, b3):
        pltpu.async_copy(mbuf.at[b3], aggsh.at[didx.at[b6]], ssems[b3],
                         add=True)

    def wait_scatter(b6, b3):
        pltpu.make_async_copy(mbuf.at[b3], aggsh.at[didx.at[b6]],
                              ssems[b3]).wait()

    def relu_chunk(k, b3):
        base = start8 + k * CH

        def body(e4, _):
            for u in range(4):
                e = e4 * 4 + u
                eg = base + e
                vf = jnp.where((eg >= start) & (eg < end),
                               jnp.float32(1.0), jnp.float32(0.0))
                for jv in range(4):
                    sl = pl.ds(jv * 16, 16)
                    mbuf[b3, e, sl] = jnp.maximum(mbuf[b3, e, sl], 0.0) * vf
            return 0

        lax.fori_loop(0, CH // 4, body, 0)

    for j in range(4):
        @pl.when(nch > j)
        def _():
            fetch_idx(j, j)

    for j in range(2):
        @pl.when(nch > j)
        def _():
            wait_idx(j)
            issue_gather(j, j)

    def grp_body(grp, _):
        for b6 in range(6):
            k = grp * 6 + b6
            b3 = b6 % 3
            pb6 = (b6 + 5) % 6
            pb3 = (b3 + 2) % 3
            nb6 = (b6 + 2) % 6
            nb3 = (b3 + 2) % 3

            @pl.when((k >= 1) & (k <= nch))
            def _():
                wait_scatter(pb6, pb3)

            @pl.when(k < nch)
            def _():
                wait_gather(b6, b3)
                relu_chunk(k, b3)
                issue_scatter(b6, b3)

            @pl.when(k + 2 < nch)
            def _():
                wait_idx(nb6)
                issue_gather(nb6, nb3)

            @pl.when(k + 4 < nch)
            def _():
                fetch_idx(k + 4, (b6 + 4) % 6)
        return 0

    lax.fori_loop(0, ngrp, grp_body, 0)

    plsc.subcore_barrier()
    pltpu.sync_copy(aggsh.at[pl.ds(sid * SP_PT, SP_PT)],
                    out.at[c, pl.ds(sid * SP_PT, SP_PT)])


_mp = pl.kernel(
    _mp_body,
    out_type=jax.ShapeDtypeStruct((2, SP_ROWS, H), jnp.float32),
    mesh=_mesh,
    compiler_params=_sc_params,
    name="sc_mp",
    scratch_types=[
        pltpu.VMEM((3, CH, H), jnp.float32),
        pltpu.VMEM((6, CH), jnp.int32),
        pltpu.VMEM((6, CH), jnp.int32),
        pltpu.VMEM((16,), jnp.float32),
        pltpu.VMEM_SHARED((SP_ROWS, H), jnp.float32),
    ] + [pltpu.SemaphoreType.DMA] * 12,
)


def _lg_body(Df, idxh, out, ibuf, dbuf, iA, iB, gsem, dA, dB):
    c = lax.axis_index("c")
    sid = lax.axis_index("s")
    wid = sid * 2 + c
    rbase = wid * (LG_PT // CH)
    isems = [iA, iB]
    dsems = [dA, dB]

    def fetch_i(g, b):
        return pltpu.async_copy(idxh.at[pl.ds(rbase + g * LG_GRP, LG_GRP)],
                                ibuf.at[b], isems[b])

    def fire_g(b):
        return [pltpu.async_copy(Df.at[ibuf.at[b, j]], dbuf.at[b, j], gsem)
                for j in range(LG_GRP)]

    pend_idx = [fetch_i(0, 0), fetch_i(1, 1)]
    pend_drain = [None, None]
    pend_idx[0].wait()
    pend_g = fire_g(0)
    for grp in range(LG_GROUPS):
        b = grp % 2
        nb = (grp + 1) % 2
        for gd in pend_g:
            gd.wait()
        if grp + 2 < LG_GROUPS:
            pend_idx[b] = fetch_i(grp + 2, b)
        if grp + 1 < LG_GROUPS:
            if pend_drain[nb] is not None:
                pend_drain[nb].wait()
            pend_idx[nb].wait()
            pend_g = fire_g(nb)
        pend_drain[b] = pltpu.async_copy(
            dbuf.at[b], out.at[pl.ds(rbase + grp * LG_GRP, LG_GRP)],
            dsems[b])
    for b in range(2):
        if pend_drain[b] is not None:
            pend_drain[b].wait()


_lg = pl.kernel(
    _lg_body,
    out_type=jax.ShapeDtypeStruct((NPL // CH, CH), jnp.float32),
    mesh=_mesh,
    compiler_params=_sc_params,
    name="sc_lg",
    scratch_types=[
        pltpu.VMEM((2, LG_GRP, CH), jnp.int32),
        pltpu.VMEM((2, LG_GRP, CH), jnp.float32),
    ] + [pltpu.SemaphoreType.DMA] * 5,
)


def _buildT_body(hr, e1r, dvr, Tout):
    hv = hr[...]
    for k in range(4):
        Tout[k] = hv + e1r[...] + k * dvr[...]


def _buildT(h, e1r, dvr):
    return pl.pallas_call(
        _buildT_body,
        grid=(NBD,),
        in_specs=[
            pl.BlockSpec((BLKD, H), lambda j: (j, 0)),
            pl.BlockSpec((1, H), lambda j: (0, 0)),
            pl.BlockSpec((1, H), lambda j: (0, 0)),
        ],
        out_specs=pl.BlockSpec((4, BLKD, H), lambda j: (0, j, 0)),
        out_shape=jax.ShapeDtypeStruct((4, N, H), jnp.float32),
    )(h, e1r, dvr)


def _dense_call(emit_T, relu_out):
    def body(*args):
        if emit_T:
            hc, agg, W1, b1, W2, b2, epsr, e1r, dvr, hout, Tout = args
        else:
            hc, agg, W1, b1, W2, b2, epsr, hout = args
        z = (1.0 + epsr[0, 0]) * hc[...] + agg[0]
        z = jnp.maximum(
            jnp.dot(z, W1[...], preferred_element_type=jnp.float32) + b1[...],
            0.0)
        z = jnp.dot(z, W2[...], preferred_element_type=jnp.float32) + b2[...]
        if relu_out:
            z = jnp.maximum(z, 0.0)
        hout[...] = z
        if emit_T:
            for k in range(4):
                Tout[k] = z + e1r[...] + k * dvr[...]

    blk = pl.BlockSpec((BLKD, H), lambda j: (j, 0))
    nb_half = HALF // BLKD
    aggspec = pl.BlockSpec((1, BLKD, H),
                           lambda j: (j // nb_half, j % nb_half, 0))
    wspec = pl.BlockSpec((H, H), lambda j: (0, 0))
    bspec = pl.BlockSpec((1, H), lambda j: (0, 0))
    escp = pl.BlockSpec((1, 1), lambda j: (0, 0))
    in_specs = [blk, aggspec, wspec, bspec, wspec, bspec, escp]
    out_specs = blk
    out_shape = jax.ShapeDtypeStruct((N, H), jnp.float32)
    if emit_T:
        in_specs = in_specs + [bspec, bspec]
        out_specs = (blk, pl.BlockSpec((4, BLKD, H), lambda j: (0, j, 0)))
        out_shape = (out_shape, jax.ShapeDtypeStruct((4, N, H), jnp.float32))
    return pl.pallas_call(
        body, grid=(NBD,), in_specs=in_specs, out_specs=out_specs,
        out_shape=out_shape)


_dense_T = _dense_call(True, True)
_dense_last = _dense_call(False, False)


def _ddot_body(xl, xg, out):
    Z = xl[...] + xg[...]
    dd = lax.dot_general(Z, Z, (((1,), (1,)), ((), ())),
                         preferred_element_type=jnp.float32)
    out[...] = dd.reshape(1, NPG, NPG)


def _ddot(xl, xg):
    return pl.pallas_call(
        _ddot_body,
        grid=(B,),
        in_specs=[
            pl.BlockSpec((NPG, H), lambda g: (g, 0)),
            pl.BlockSpec((NPG, H), lambda g: (g, 0)),
        ],
        out_specs=pl.BlockSpec((1, NPG, NPG), lambda g: (g, 0, 0)),
        out_shape=jax.ShapeDtypeStruct((B, NPG, NPG), jnp.float32),
    )(xl, xg)


def _loss_body(d, g, acc):
    pid = pl.program_id(0)

    @pl.when(pid == 0)
    def _():
        acc[...] = jnp.zeros((8, 64), jnp.float32)

    dv = d[0, 0, :]
    sg = 1.0 / (1.0 + jnp.exp(-dv))
    tpos = -jnp.log(sg + EPS)
    tneg = -jnp.log(1.0 - sg + EPS)
    posf = jnp.where(pid < NBL // 2, jnp.float32(1.0), jnp.float32(0.0))
    t = posf * tpos + (1.0 - posf) * tneg
    oh = (g[0, 0, :][:, None] == lax.broadcasted_iota(jnp.int32, (BLKL, 64), 1)
          ).astype(jnp.float32)
    rows = lax.broadcasted_iota(jnp.int32, (8, BLKL), 0)
    tb = t[None, :]
    M = jnp.where(rows == 0, tb * posf,
                  jnp.where(rows == 1, posf,
                            jnp.where(rows == 2, tb * (1.0 - posf), 0.0)))
    acc[...] += jnp.dot(M, oh, preferred_element_type=jnp.float32)


def _loss(dmat, gmat):
    return pl.pallas_call(
        _loss_body,
        grid=(NBL,),
        in_specs=[
            pl.BlockSpec((1, 1, BLKL), lambda j: (j, 0, 0)),
            pl.BlockSpec((1, 1, BLKL), lambda j: (j, 0, 0)),
        ],
        out_specs=pl.BlockSpec((8, 64), lambda j: (0, 0)),
        out_shape=jax.ShapeDtypeStruct((8, 64), jnp.float32),
    )(dmat, gmat)


def _fin_body(xg, accr, W1r, b1r, gr, br, Wc, bc, louts, recout):
    pooled = jnp.sum(xg[...].reshape(B, NPG, H), axis=1)
    o = jnp.dot(pooled, W1r[...], preferred_element_type=jnp.float32) + b1r[...]
    mu = jnp.mean(o, axis=0, keepdims=True)
    var = jnp.mean((o - mu) ** 2, axis=0, keepdims=True)
    o = gr[...] * (o - mu) / jnp.sqrt(var + 1e-5) + br[...]
    o = jnp.maximum(o, 0.0)
    louts[...] = jnp.dot(o, Wc[...], preferred_element_type=jnp.float32) + bc[...]
    a = accr[...]
    lanes = lax.broadcasted_iota(jnp.int32, (1, 64), 1)
    lossv = jnp.where(lanes < B, (a[0:1, :] + a[2:3, :]) / a[1:2, :], 0.0)
    recout[...] = jnp.sum(lossv, keepdims=True) / B


def _final(xg, acc, W1r, b1r, gr, br, Wc, bc):
    return pl.pallas_call(
        _fin_body,
        out_shape=(jax.ShapeDtypeStruct((B, 128), jnp.float32),
                   jax.ShapeDtypeStruct((1, 1), jnp.float32)),
    )(xg, acc, W1r, b1r, gr, br, Wc, bc)


def kernel(x, edge_index, batch, edge_attr, neg_edge_index, params):
    p = params
    src = edge_index[0].astype(jnp.int32)
    dst = edge_index[1].astype(jnp.int32)
    s = jnp.sum(edge_attr, axis=1).astype(jnp.int32)
    g_e = dst // NPG
    E1 = p["edge_emb"][1]
    Dv = (p["edge_emb"][2] - E1) / 3.0
    e1r = E1.reshape(1, H)
    dvr = Dv.reshape(1, H)

    xi = x.astype(jnp.int32) + jnp.arange(IN_CH, dtype=jnp.int32)[None, :] * EMD
    xiT = jnp.pad(xi.T, ((0, 0), (0, NP_ENC - N)))
    emb = p["atom_emb"].reshape(IN_CH * EMD, H)
    h = _enc(xiT, emb)[:N]

    gidx = jnp.pad(s * N + src, (0, EP - E))
    didx = jnp.pad(dst % HALF, (0, EP - E))
    esplit = jnp.searchsorted(g_e, 25).astype(jnp.float32)
    esv = jnp.zeros((16,), jnp.float32).at[0].set(esplit)
    zer = jnp.zeros((SP_PT, H), jnp.float32)

    def mp(Tfull):
        return _mp(Tfull.reshape(4 * N, H), gidx, didx, esv, zer)

    T1 = _buildT(h, e1r, dvr)
    agg0 = mp(T1)

    def run_net(plist):
        p0, p1 = plist
        hc1, T2 = _dense_T(h, agg0, p0["W1"], p0["b1"].reshape(1, H),
                           p0["W2"], p0["b2"].reshape(1, H),
                           p0["eps"].reshape(1, 1), e1r, dvr)
        agg1 = mp(T2)
        return _dense_last(hc1, agg1, p1["W1"], p1["b1"].reshape(1, H),
                           p1["W2"], p1["b2"].reshape(1, H),
                           p1["eps"].reshape(1, 1))

    x_local = run_net(p["local"])
    x_global = run_net(p["glob"])

    Df = _ddot(x_local, x_global).reshape(B * NPG * NPG)
    pos_idx = g_e * (NPG * NPG) + (src % NPG) * NPG + (dst % NPG)
    ns = neg_edge_index[0].astype(jnp.int32)
    nd = neg_edge_index[1].astype(jnp.int32)
    neg_idx = (ns // NPG) * (NPG * NPG) + (ns % NPG) * NPG + (nd % NPG)
    idx_all = jnp.pad(jnp.concatenate([pos_idx, neg_idx]),
                      (0, NPL - 2 * E)).reshape(NPL // CH, CH)
    d_all = _lg(Df, idx_all).reshape(NPL)
    dmat = d_all[:2 * E].reshape(NBL, 1, BLKL)
    gmat = jnp.concatenate([g_e, g_e]).reshape(NBL, 1, BLKL)
    acc = _loss(dmat, gmat)

    logits_pad, rec = _final(
        x_global, acc, p["W_lin1"], p["b_lin1"].reshape(1, H),
        p["bn_gamma"].reshape(1, H), p["bn_beta"].reshape(1, H),
        jnp.pad(p["W_clf"], ((0, 0), (0, 128 - OUT))),
        jnp.pad(p["b_clf"], (0, 128 - OUT)).reshape(1, 128))
    return (logits_pad[:, :OUT], x_local, x_global, rec.reshape(()))


# speedup vs baseline: 16.9373x; 1.2429x over previous
import jax
import jax.numpy as jnp
from jax import lax
from jax.experimental import pallas as pl
from jax.experimental.pallas import tpu as pltpu
from jax.experimental.pallas import tpu_sc as plsc

N = 50000
B = 50
NPG = 1000
E = 800000
H = 64
EMD = 200
IN_CH = 9
OUT = 10
EPS = 1e-15
HALF = N // 2
CH = 128

ENC_CHUNKS = 13
ENC_PT = ENC_CHUNKS * CH
NP_ENC = 32 * ENC_PT

SP_PT = 1600
SP_ROWS = 16 * SP_PT
EP = E + 2 * CH

LG_GRP = 16
LG_GROUPS = 25
LG_PT = LG_GROUPS * LG_GRP * CH
NPL = 32 * LG_PT

BLKL = 16000
NBL = 2 * E // BLKL
BLKD = 1000
NBD = N // BLKD

_mesh = plsc.VectorSubcoreMesh(core_axis_name="c", subcore_axis_name="s")
_sc_params = pltpu.CompilerParams(use_tc_tiling_on_sc=False,
                                  needs_layout_passes=False)


ECH = 64
ENC_NCH = ENC_PT // ECH


EMB_PAD = 1824


def _enc_body(xiT, emb, out, ibuf, gbuf, hbuf, emb_sp, i0, i1, g0, g1, d0, d1):
    c = lax.axis_index("c")
    sid = lax.axis_index("s")
    wid = sid * 2 + c
    isems = [i0, i1]
    gsems = [g0, g1]
    dsems = [d0, d1]

    rows = EMB_PAD // 16
    pltpu.sync_copy(emb.at[pl.ds(sid * rows, rows)],
                    emb_sp.at[pl.ds(sid * rows, rows)])
    plsc.subcore_barrier()

    def issue_idx(k, b):
        nb = wid * ENC_PT + k * ECH
        return pltpu.async_copy(xiT.at[:, pl.ds(nb, ECH)], ibuf.at[b],
                                isems[b])

    def issue_gathers(b):
        return [pltpu.async_copy(emb_sp.at[ibuf.at[b, i]], gbuf.at[b, i],
                                 gsems[b]) for i in range(IN_CH)]

    idxd = [issue_idx(0, 0), issue_idx(1, 1)]
    gatd = [None, None]
    hd = [None, None]
    idxd[0].wait()
    gatd[0] = issue_gathers(0)
    for k in range(ENC_NCH):
        b = k % 2
        nb = wid * ENC_PT + k * ECH
        if k + 1 < ENC_NCH:
            idxd[(k + 1) % 2].wait()
            gatd[(k + 1) % 2] = issue_gathers((k + 1) % 2)
        for d in gatd[b]:
            d.wait()
        if k + 2 < ENC_NCH:
            idxd[b] = issue_idx(k + 2, b)
        if hd[b] is not None:
            hd[b].wait()

        def body(e2, _):
            for u in range(2):
                e = e2 * 2 + u
                for jv in range(4):
                    sl = pl.ds(jv * 16, 16)
                    acc = gbuf[b, 0, e, sl]
                    for i in range(1, IN_CH):
                        acc = acc + gbuf[b, i, e, sl]
                    hbuf[b, e, sl] = acc
            return 0

        lax.fori_loop(0, ECH // 2, body, 0)
        hd[b] = pltpu.async_copy(hbuf.at[b], out.at[pl.ds(nb, ECH)], dsems[b])
    for b in range(2):
        if hd[b] is not None:
            hd[b].wait()


_enc = pl.kernel(
    _enc_body,
    out_type=jax.ShapeDtypeStruct((NP_ENC, H), jnp.float32),
    mesh=_mesh,
    compiler_params=_sc_params,
    name="sc_enc",
    scratch_types=[
        pltpu.VMEM((2, IN_CH, ECH), jnp.int32),
        pltpu.VMEM((2, IN_CH, ECH, H), jnp.float32),
        pltpu.VMEM((2, ECH, H), jnp.float32),
        pltpu.VMEM_SHARED((EMB_PAD, H), jnp.float32),
    ] + [pltpu.SemaphoreType.DMA] * 6,
)


def _mp_body(T, gidxh, didxh, espl, zer, out, mbuf, gidx, didx, esv, aggsh,
             i0, i1, i2, i3, i4, i5, g0, g1, g2, s0, s1, s2):
    isems = [i0, i1, i2, i3, i4, i5]
    gsems = [g0, g1, g2]
    ssems = [s0, s1, s2]
    c = lax.axis_index("c")
    sid = lax.axis_index("s")

    pltpu.sync_copy(zer, aggsh.at[pl.ds(sid * SP_PT, SP_PT)])
    pltpu.sync_copy(espl, esv)
    lanes = lax.iota(jnp.int32, 16)
    esplit_f = jnp.sum(jnp.where(lanes == 0, esv[...], jnp.float32(0.0)))
    esplit = esplit_f.astype(jnp.int32)
    lo = jnp.where(c == 0, 0, esplit)
    hi = jnp.where(c == 0, esplit, E)
    per = (hi - lo + 15) // 16
    start = lo + sid * per
    end = jnp.minimum(start + per, hi)
    start8 = (start // 8) * 8
    nch = jnp.maximum((end - start8 + CH - 1) // CH, 0)
    ngrp = (nch + 7) // 6

    plsc.subcore_barrier()

    def fetch_idx(k, b6):
        base = start8 + k * CH
        pltpu.async_copy(gidxh.at[pl.ds(base, CH)], gidx.at[b6], isems[b6])
        pltpu.async_copy(didxh.at[pl.ds(base, CH)], didx.at[b6], isems[b6])

    def wait_idx(b6):
        pltpu.make_async_copy(gidxh.at[pl.ds(0, CH)], gidx.at[b6],
                              isems[b6]).wait()
        pltpu.make_async_copy(didxh.at[pl.ds(0, CH)], didx.at[b6],
                              isems[b6]).wait()

    def issue_gather(b6, b3):
        pltpu.async_copy(T.at[gidx.at[b6]], mbuf.at[b3], gsems[b3])

    def wait_gather(b6, b3):
        pltpu.make_async_copy(T.at[gidx.at[b6]], mbuf.at[b3],
                              gsems[b3]).wait()

    def issue_scatter(b6, b3):
        pltpu.async_copy(mbuf.at[b3], aggsh.at[didx.at[b6]], ssems[b3],
                         add=True)

    def wait_scatter(b6, b3):
        pltpu.make_async_copy(mbuf.at[b3], aggsh.at[didx.at[b6]],
                              ssems[b3]).wait()

    def relu_chunk(k, b3):
        base = start8 + k * CH

        def body(e4, _):
            for u in range(4):
                e = e4 * 4 + u
                eg = base + e
                vf = jnp.where((eg >= start) & (eg < end),
                               jnp.float32(1.0), jnp.float32(0.0))
                for jv in range(4):
                    sl = pl.ds(jv * 16, 16)
                    mbuf[b3, e, sl] = jnp.maximum(mbuf[b3, e, sl], 0.0) * vf
            return 0

        lax.fori_loop(0, CH // 4, body, 0)

    for j in range(4):
        @pl.when(nch > j)
        def _():
            fetch_idx(j, j)

    for j in range(2):
        @pl.when(nch > j)
        def _():
            wait_idx(j)
            issue_gather(j, j)

    def grp_body(grp, _):
        for b6 in range(6):
            k = grp * 6 + b6
            b3 = b6 % 3
            pb6 = (b6 + 5) % 6
            pb3 = (b3 + 2) % 3
            nb6 = (b6 + 2) % 6
            nb3 = (b3 + 2) % 3

            @pl.when((k >= 1) & (k <= nch))
            def _():
                wait_scatter(pb6, pb3)

            @pl.when(k < nch)
            def _():
                wait_gather(b6, b3)
                relu_chunk(k, b3)
                issue_scatter(b6, b3)

            @pl.when(k + 2 < nch)
            def _():
                wait_idx(nb6)
                issue_gather(nb6, nb3)

            @pl.when(k + 4 < nch)
            def _():
                fetch_idx(k + 4, (b6 + 4) % 6)
        return 0

    lax.fori_loop(0, ngrp, grp_body, 0)

    plsc.subcore_barrier()
    pltpu.sync_copy(aggsh.at[pl.ds(sid * SP_PT, SP_PT)],
                    out.at[c, pl.ds(sid * SP_PT, SP_PT)])


_mp = pl.kernel(
    _mp_body,
    out_type=jax.ShapeDtypeStruct((2, SP_ROWS, H), jnp.float32),
    mesh=_mesh,
    compiler_params=_sc_params,
    name="sc_mp",
    scratch_types=[
        pltpu.VMEM((3, CH, H), jnp.float32),
        pltpu.VMEM((6, CH), jnp.int32),
        pltpu.VMEM((6, CH), jnp.int32),
        pltpu.VMEM((16,), jnp.float32),
        pltpu.VMEM_SHARED((SP_ROWS, H), jnp.float32),
    ] + [pltpu.SemaphoreType.DMA] * 12,
)


def _lg_body(Df, idxh, out, ibuf, dbuf, iA, iB, gsem, dA, dB):
    c = lax.axis_index("c")
    sid = lax.axis_index("s")
    wid = sid * 2 + c
    rbase = wid * (LG_PT // CH)
    isems = [iA, iB]
    dsems = [dA, dB]

    def fetch_i(g, b):
        return pltpu.async_copy(idxh.at[pl.ds(rbase + g * LG_GRP, LG_GRP)],
                                ibuf.at[b], isems[b])

    def fire_g(b):
        return [pltpu.async_copy(Df.at[ibuf.at[b, j]], dbuf.at[b, j], gsem)
                for j in range(LG_GRP)]

    pend_idx = [fetch_i(0, 0), fetch_i(1, 1)]
    pend_drain = [None, None]
    pend_idx[0].wait()
    pend_g = fire_g(0)
    for grp in range(LG_GROUPS):
        b = grp % 2
        nb = (grp + 1) % 2
        for gd in pend_g:
            gd.wait()
        if grp + 2 < LG_GROUPS:
            pend_idx[b] = fetch_i(grp + 2, b)
        if grp + 1 < LG_GROUPS:
            if pend_drain[nb] is not None:
                pend_drain[nb].wait()
            pend_idx[nb].wait()
            pend_g = fire_g(nb)
        pend_drain[b] = pltpu.async_copy(
            dbuf.at[b], out.at[pl.ds(rbase + grp * LG_GRP, LG_GRP)],
            dsems[b])
    for b in range(2):
        if pend_drain[b] is not None:
            pend_drain[b].wait()


_lg = pl.kernel(
    _lg_body,
    out_type=jax.ShapeDtypeStruct((NPL // CH, CH), jnp.float32),
    mesh=_mesh,
    compiler_params=_sc_params,
    name="sc_lg",
    scratch_types=[
        pltpu.VMEM((2, LG_GRP, CH), jnp.int32),
        pltpu.VMEM((2, LG_GRP, CH), jnp.float32),
    ] + [pltpu.SemaphoreType.DMA] * 5,
)


def _buildT_body(hr, e1r, dvr, Tout):
    hv = hr[...]
    for k in range(4):
        Tout[k] = hv + e1r[...] + k * dvr[...]


def _buildT(h, e1r, dvr):
    return pl.pallas_call(
        _buildT_body,
        grid=(NBD,),
        in_specs=[
            pl.BlockSpec((BLKD, H), lambda j: (j, 0)),
            pl.BlockSpec((1, H), lambda j: (0, 0)),
            pl.BlockSpec((1, H), lambda j: (0, 0)),
        ],
        out_specs=pl.BlockSpec((4, BLKD, H), lambda j: (0, j, 0)),
        out_shape=jax.ShapeDtypeStruct((4, N, H), jnp.float32),
    )(h, e1r, dvr)


def _dense_call(emit_T, relu_out):
    def body(*args):
        if emit_T:
            hc, agg, W1, b1, W2, b2, epsr, e1r, dvr, hout, Tout = args
        else:
            hc, agg, W1, b1, W2, b2, epsr, hout = args
        z = (1.0 + epsr[0, 0]) * hc[...] + agg[0]
        z = jnp.maximum(
            jnp.dot(z, W1[...], preferred_element_type=jnp.float32) + b1[...],
            0.0)
        z = jnp.dot(z, W2[...], preferred_element_type=jnp.float32) + b2[...]
        if relu_out:
            z = jnp.maximum(z, 0.0)
        hout[...] = z
        if emit_T:
            for k in range(4):
                Tout[k] = z + e1r[...] + k * dvr[...]

    blk = pl.BlockSpec((BLKD, H), lambda j: (j, 0))
    nb_half = HALF // BLKD
    aggspec = pl.BlockSpec((1, BLKD, H),
                           lambda j: (j // nb_half, j % nb_half, 0))
    wspec = pl.BlockSpec((H, H), lambda j: (0, 0))
    bspec = pl.BlockSpec((1, H), lambda j: (0, 0))
    escp = pl.BlockSpec((1, 1), lambda j: (0, 0))
    in_specs = [blk, aggspec, wspec, bspec, wspec, bspec, escp]
    out_specs = blk
    out_shape = jax.ShapeDtypeStruct((N, H), jnp.float32)
    if emit_T:
        in_specs = in_specs + [bspec, bspec]
        out_specs = (blk, pl.BlockSpec((4, BLKD, H), lambda j: (0, j, 0)))
        out_shape = (out_shape, jax.ShapeDtypeStruct((4, N, H), jnp.float32))
    return pl.pallas_call(
        body, grid=(NBD,), in_specs=in_specs, out_specs=out_specs,
        out_shape=out_shape)


_dense_T = _dense_call(True, True)
_dense_last = _dense_call(False, False)


def _ddot_body(xl, xg, out):
    Z = xl[...] + xg[...]
    dd = lax.dot_general(Z, Z, (((1,), (1,)), ((), ())),
                         preferred_element_type=jnp.float32)
    out[...] = dd.reshape(1, NPG, NPG)


def _ddot(xl, xg):
    return pl.pallas_call(
        _ddot_body,
        grid=(B,),
        in_specs=[
            pl.BlockSpec((NPG, H), lambda g: (g, 0)),
            pl.BlockSpec((NPG, H), lambda g: (g, 0)),
        ],
        out_specs=pl.BlockSpec((1, NPG, NPG), lambda g: (g, 0, 0)),
        out_shape=jax.ShapeDtypeStruct((B, NPG, NPG), jnp.float32),
    )(xl, xg)


def _loss_body(d, g, acc):
    pid = pl.program_id(0)

    @pl.when(pid == 0)
    def _():
        acc[...] = jnp.zeros((8, 64), jnp.float32)

    sgp = 1.0 / (1.0 + jnp.exp(-d[0, 0, :]))
    sgn = 1.0 / (1.0 + jnp.exp(-d[0, 1, :]))
    tpos = -jnp.log(sgp + EPS)
    tneg = -jnp.log(1.0 - sgn + EPS)
    oh = (g[0, 0, :][:, None] == lax.broadcasted_iota(jnp.int32, (BLKL, 64), 1)
          ).astype(jnp.float32)
    rows = lax.broadcasted_iota(jnp.int32, (8, BLKL), 0)
    M = jnp.where(rows == 0, tpos[None, :],
                  jnp.where(rows == 1, jnp.float32(1.0),
                            jnp.where(rows == 2, tneg[None, :], 0.0)))
    acc[...] += jnp.dot(M, oh, preferred_element_type=jnp.float32)


def _loss(dmat, gmat):
    return pl.pallas_call(
        _loss_body,
        grid=(B,),
        in_specs=[
            pl.BlockSpec((1, 2, BLKL), lambda j: (j, 0, 0)),
            pl.BlockSpec((1, 1, BLKL), lambda j: (j, 0, 0)),
        ],
        out_specs=pl.BlockSpec((8, 64), lambda j: (0, 0)),
        out_shape=jax.ShapeDtypeStruct((8, 64), jnp.float32),
    )(dmat, gmat)


def _fin_body(xg, accr, W1r, b1r, gr, br, Wc, bc, louts, recout):
    pooled = jnp.sum(xg[...].reshape(B, NPG, H), axis=1)
    o = jnp.dot(pooled, W1r[...], preferred_element_type=jnp.float32) + b1r[...]
    mu = jnp.mean(o, axis=0, keepdims=True)
    var = jnp.mean((o - mu) ** 2, axis=0, keepdims=True)
    o = gr[...] * (o - mu) / jnp.sqrt(var + 1e-5) + br[...]
    o = jnp.maximum(o, 0.0)
    louts[...] = jnp.dot(o, Wc[...], preferred_element_type=jnp.float32) + bc[...]
    a = accr[...]
    lanes = lax.broadcasted_iota(jnp.int32, (1, 64), 1)
    lossv = jnp.where(lanes < B, (a[0:1, :] + a[2:3, :]) / a[1:2, :], 0.0)
    recout[...] = jnp.sum(lossv, keepdims=True) / B


def _final(xg, acc, W1r, b1r, gr, br, Wc, bc):
    return pl.pallas_call(
        _fin_body,
        out_shape=(jax.ShapeDtypeStruct((B, 128), jnp.float32),
                   jax.ShapeDtypeStruct((1, 1), jnp.float32)),
    )(xg, acc, W1r, b1r, gr, br, Wc, bc)


def kernel(x, edge_index, batch, edge_attr, neg_edge_index, params):
    p = params
    src = edge_index[0].astype(jnp.int32)
    dst = edge_index[1].astype(jnp.int32)
    s = jnp.sum(edge_attr, axis=1).astype(jnp.int32)
    g_e = dst // NPG
    E1 = p["edge_emb"][1]
    Dv = (p["edge_emb"][2] - E1) / 3.0
    e1r = E1.reshape(1, H)
    dvr = Dv.reshape(1, H)

    xi = x.astype(jnp.int32) + jnp.arange(IN_CH, dtype=jnp.int32)[None, :] * EMD
    xiT = jnp.pad(xi.T, ((0, 0), (0, NP_ENC - N)))
    emb = jnp.pad(p["atom_emb"].reshape(IN_CH * EMD, H),
                  ((0, EMB_PAD - IN_CH * EMD), (0, 0)))
    h = _enc(xiT, emb)[:N]

    gidx = jnp.pad(s * N + src, (0, EP - E))
    didx = jnp.pad(dst % HALF, (0, EP - E))
    esplit = jnp.searchsorted(g_e, 25).astype(jnp.float32)
    esv = jnp.zeros((16,), jnp.float32).at[0].set(esplit)
    zer = jnp.zeros((SP_PT, H), jnp.float32)

    def mp(Tfull):
        return _mp(Tfull.reshape(4 * N, H), gidx, didx, esv, zer)

    T1 = _buildT(h, e1r, dvr)
    agg0 = mp(T1)

    def run_net(plist):
        p0, p1 = plist
        hc1, T2 = _dense_T(h, agg0, p0["W1"], p0["b1"].reshape(1, H),
                           p0["W2"], p0["b2"].reshape(1, H),
                           p0["eps"].reshape(1, 1), e1r, dvr)
        agg1 = mp(T2)
        return _dense_last(hc1, agg1, p1["W1"], p1["b1"].reshape(1, H),
                           p1["W2"], p1["b2"].reshape(1, H),
                           p1["eps"].reshape(1, 1))

    x_local = run_net(p["local"])
    x_global = run_net(p["glob"])

    Df = _ddot(x_local, x_global).reshape(B * NPG * NPG)
    pos_idx = g_e * (NPG * NPG) + (src % NPG) * NPG + (dst % NPG)
    ns = neg_edge_index[0].astype(jnp.int32)
    nd = neg_edge_index[1].astype(jnp.int32)
    neg_idx = (ns // NPG) * (NPG * NPG) + (ns % NPG) * NPG + (nd % NPG)
    idx_all = jnp.pad(jnp.concatenate([pos_idx, neg_idx]),
                      (0, NPL - 2 * E)).reshape(NPL // CH, CH)
    d_all = _lg(Df, idx_all).reshape(NPL)
    dmat = jnp.stack([d_all[:E].reshape(B, BLKL),
                      d_all[E:2 * E].reshape(B, BLKL)], axis=1)
    gmat = g_e.reshape(B, 1, BLKL)
    acc = _loss(dmat, gmat)

    logits_pad, rec = _final(
        x_global, acc, p["W_lin1"], p["b_lin1"].reshape(1, H),
        p["bn_gamma"].reshape(1, H), p["bn_beta"].reshape(1, H),
        jnp.pad(p["W_clf"], ((0, 0), (0, 128 - OUT))),
        jnp.pad(p["b_clf"], (0, 128 - OUT)).reshape(1, 128))
    return (logits_pad[:, :OUT], x_local, x_global, rec.reshape(()))
